# baseline (speedup 1.0000x reference)
import functools

import jax
import jax.numpy as jnp
from jax import lax
from jax.experimental import pallas as pl
from jax.experimental.pallas import tpu as pltpu
from jax.experimental.pallas import tpu_sc as plsc

N = 10000
E = 320000
D = 128
NP = 10240
BLK = 1024
NBLK = NP // BLK

NC = 2
NS = 16
NW = NC * NS
EPW = E // NW
CHUNK = 80
NCHUNK = EPW // CHUNK
DCHUNK = 1000
NDCHUNK = EPW // DCHUNK
RPT = NP // NS
ZR = 64


def _ln1_body(x_ref, w_ref, b_ref, o_ref):
    x = x_ref[...]
    mu = jnp.mean(x, axis=1, keepdims=True)
    d = x - mu
    var = jnp.mean(d * d, axis=1, keepdims=True)
    o_ref[...] = d / jnp.sqrt(var + 1e-5) * w_ref[...] + b_ref[...]


def _ln1(x, w, b):
    return pl.pallas_call(
        _ln1_body,
        grid=(NBLK,),
        in_specs=[
            pl.BlockSpec((BLK, D), lambda i: (i, 0)),
            pl.BlockSpec((1, D), lambda i: (0, 0)),
            pl.BlockSpec((1, D), lambda i: (0, 0)),
        ],
        out_specs=pl.BlockSpec((BLK, D), lambda i: (i, 0)),
        out_shape=jax.ShapeDtypeStruct((NP, D), jnp.float32),
    )(x, w, b)


NB = 4


def _sc_agg_body(h_hbm, src_hbm, dst_hbm,
                 sum_a, sum_b, deg_a, deg_b,
                 acc_sp, deg_sp,
                 src0, src1, src2, src3, dst0, dst1, dst2, dst3,
                 rows0, rows1, rows2, rows3,
                 dst2_v, ones_v, zflat_v, sem0, sem1, sem2, sem3):
    cid = lax.axis_index("c")
    sid = lax.axis_index("s")
    wid = cid * NS + sid

    zeros16 = jnp.zeros((16,), jnp.float32)

    @pl.loop(0, (CHUNK * D) // 16)
    def _zr(i):
        rows0[i // 8, pl.ds((i % 8) * 16, 16)] = zeros16

    @pl.loop(0, RPT // 16)
    def _zf(i):
        zflat_v[pl.ds(i * 16, 16)] = zeros16

    @pl.loop(0, DCHUNK // 16)
    def _on(i):
        ones_v[pl.ds(i * 16, 16)] = jnp.ones((16,), jnp.float32)

    for j in range(RPT // CHUNK):
        pltpu.sync_copy(rows0, acc_sp.at[pl.ds(sid * RPT + j * CHUNK, CHUNK)])
    pltpu.sync_copy(zflat_v, deg_sp.at[pl.ds(sid * RPT, RPT)])
    plsc.subcore_barrier()

    srcs = (src0, src1, src2, src3)
    dsts = (dst0, dst1, dst2, dst3)
    rows = (rows0, rows1, rows2, rows3)
    sems = (sem0, sem1, sem2, sem3)
    ebase = wid * EPW

    def _fetch(c, b):
        pltpu.sync_copy(src_hbm.at[pl.ds(ebase + c * CHUNK, CHUNK)], srcs[b])
        pltpu.sync_copy(dst_hbm.at[pl.ds(ebase + c * CHUNK, CHUNK)], dsts[b])
        pltpu.async_copy(h_hbm.at[srcs[b]], rows[b], sems[b])

    for b in range(NB):
        _fetch(b, b)

    NMAIN = (NCHUNK // NB) * NB - NB

    @pl.loop(0, NMAIN, step=NB)
    def _step(k):
        for b in range(NB):
            pltpu.make_async_copy(h_hbm.at[srcs[b]], rows[b], sems[b]).wait()
            pltpu.sync_copy(rows[b], acc_sp.at[dsts[b]], add=True)
            _fetch(k + b + NB, b)

    for c in range(NMAIN, NCHUNK):
        b = c % NB
        pltpu.make_async_copy(h_hbm.at[srcs[b]], rows[b], sems[b]).wait()
        pltpu.sync_copy(rows[b], acc_sp.at[dsts[b]], add=True)
        if c + NB < NCHUNK:
            _fetch(c + NB, b)

    @pl.loop(0, NDCHUNK)
    def _dstep(k):
        base = wid * EPW + k * DCHUNK
        pltpu.sync_copy(dst_hbm.at[pl.ds(base, DCHUNK)], dst2_v)
        pltpu.sync_copy(ones_v, deg_sp.at[dst2_v], add=True)

    plsc.subcore_barrier()

    row0 = sid * RPT

    @pl.when(cid == 0)
    def _():
        pltpu.sync_copy(acc_sp.at[pl.ds(row0, RPT)], sum_a.at[pl.ds(row0, RPT)])
        pltpu.sync_copy(deg_sp.at[pl.ds(row0, RPT)], deg_a.at[pl.ds(row0, RPT)])

    @pl.when(cid == 1)
    def _():
        pltpu.sync_copy(acc_sp.at[pl.ds(row0, RPT)], sum_b.at[pl.ds(row0, RPT)])
        pltpu.sync_copy(deg_sp.at[pl.ds(row0, RPT)], deg_b.at[pl.ds(row0, RPT)])


@functools.cache
def _make_sc_agg():
  return pl.kernel(
    _sc_agg_body,
    out_type=(
        jax.ShapeDtypeStruct((NP, D), jnp.float32),
        jax.ShapeDtypeStruct((NP, D), jnp.float32),
        jax.ShapeDtypeStruct((NP,), jnp.float32),
        jax.ShapeDtypeStruct((NP,), jnp.float32),
    ),
    mesh=plsc.VectorSubcoreMesh(
        core_axis_name="c", subcore_axis_name="s",
        num_cores=NC, num_subcores=NS),
    scratch_types=[
        pltpu.VMEM_SHARED((NP, D), jnp.float32),
        pltpu.VMEM_SHARED((NP,), jnp.float32),
        pltpu.VMEM((CHUNK,), jnp.int32),
        pltpu.VMEM((CHUNK,), jnp.int32),
        pltpu.VMEM((CHUNK,), jnp.int32),
        pltpu.VMEM((CHUNK,), jnp.int32),
        pltpu.VMEM((CHUNK,), jnp.int32),
        pltpu.VMEM((CHUNK,), jnp.int32),
        pltpu.VMEM((CHUNK,), jnp.int32),
        pltpu.VMEM((CHUNK,), jnp.int32),
        pltpu.VMEM((CHUNK, D), jnp.float32),
        pltpu.VMEM((CHUNK, D), jnp.float32),
        pltpu.VMEM((CHUNK, D), jnp.float32),
        pltpu.VMEM((CHUNK, D), jnp.float32),
        pltpu.VMEM((DCHUNK,), jnp.int32),
        pltpu.VMEM((DCHUNK,), jnp.float32),
        pltpu.VMEM((RPT,), jnp.float32),
        pltpu.SemaphoreType.DMA,
        pltpu.SemaphoreType.DMA,
        pltpu.SemaphoreType.DMA,
        pltpu.SemaphoreType.DMA,
    ],
  )


def _tail_body(h_ref, pa_ref, pb_ref, da_ref, db_ref,
               ws_ref, wn_ref, wl_ref, bs_ref, bl_ref,
               l2w_ref, l2b_ref, o_ref):
    f32 = jnp.float32
    h = h_ref[...]
    summed = pa_ref[...] + pb_ref[...]
    deg = da_ref[0] + db_ref[0]
    deg_t = lax.dot_general(deg, jnp.ones((1, D), f32),
                            (((0,), (0,)), ((), ())),
                            preferred_element_type=f32)
    h_neigh = summed / jnp.maximum(deg_t, 1.0)
    rst = (lax.dot_general(h, ws_ref[...], (((1,), (1,)), ((), ())),
                           preferred_element_type=f32)
           + lax.dot_general(h_neigh, wn_ref[...], (((1,), (1,)), ((), ())),
                             preferred_element_type=f32)
           + bs_ref[...])
    h2 = rst + h
    mu = jnp.mean(h2, axis=1, keepdims=True)
    d = h2 - mu
    var = jnp.mean(d * d, axis=1, keepdims=True)
    hn = d / jnp.sqrt(var + 1e-5) * l2w_ref[...] + l2b_ref[...]
    lin = lax.dot_general(hn, wl_ref[...], (((1,), (1,)), ((), ())),
                          preferred_element_type=f32) + bl_ref[...]
    o_ref[...] = jnp.where(lin > 0, lin, jnp.exp(jnp.minimum(lin, 0.0)) - 1.0) + hn


def _tail(h, pa, pb, da, db, w_self, w_neigh, w_lin, b_sage, b_lin,
          ln2_w, ln2_b):
    row = pl.BlockSpec((BLK, D), lambda i: (i, 0))
    mat = pl.BlockSpec((D, D), lambda i: (0, 0))
    vec = pl.BlockSpec((1, D), lambda i: (0, 0))
    dspec = pl.BlockSpec((1, 1, BLK), lambda i: (i, 0, 0))
    return pl.pallas_call(
        _tail_body,
        grid=(NBLK,),
        in_specs=[row, row, row, dspec, dspec, mat, mat, mat, vec, vec,
                  vec, vec],
        out_specs=row,
        out_shape=jax.ShapeDtypeStruct((N, D), jnp.float32),
    )(h, pa, pb, da, db, w_self, w_neigh, w_lin, b_sage, b_lin, ln2_w, ln2_b)


@jax.jit
def kernel(x, ln1_w, ln1_b, w_self, w_neigh, b_sage, ln2_w, ln2_b, w_lin,
           b_lin, edge_index):
    h = _ln1(x, ln1_w.reshape(1, D), ln1_b.reshape(1, D))
    src = edge_index[0]
    dst = edge_index[1]
    sum_a, sum_b, deg_a, deg_b = _make_sc_agg()(h, src, dst)
    return _tail(h, sum_a, sum_b,
                 deg_a.reshape(NBLK, 1, BLK), deg_b.reshape(NBLK, 1, BLK),
                 w_self, w_neigh, w_lin,
                 b_sage.reshape(1, D), b_lin.reshape(1, D),
                 ln2_w.reshape(1, D), ln2_b.reshape(1, D))


# speedup vs baseline: 1.0112x; 1.0112x over previous
import functools

import jax
import jax.numpy as jnp
from jax import lax
from jax.experimental import pallas as pl
from jax.experimental.pallas import tpu as pltpu
from jax.experimental.pallas import tpu_sc as plsc

N = 10000
E = 320000
D = 128
NP = 10240
BLK = 1024
NBLK = NP // BLK

NC = 2
NS = 16
NW = NC * NS
EPW = E // NW
CHUNK = 80
NCHUNK = EPW // CHUNK
DCHUNK = 2000
NDCHUNK = EPW // DCHUNK
RPT = NP // NS
ZR = 64


def _ln1_body(x_ref, w_ref, b_ref, o_ref):
    x = x_ref[...]
    mu = jnp.mean(x, axis=1, keepdims=True)
    d = x - mu
    var = jnp.mean(d * d, axis=1, keepdims=True)
    o_ref[...] = d / jnp.sqrt(var + 1e-5) * w_ref[...] + b_ref[...]


def _ln1(x, w, b):
    return pl.pallas_call(
        _ln1_body,
        grid=(NBLK,),
        in_specs=[
            pl.BlockSpec((BLK, D), lambda i: (i, 0)),
            pl.BlockSpec((1, D), lambda i: (0, 0)),
            pl.BlockSpec((1, D), lambda i: (0, 0)),
        ],
        out_specs=pl.BlockSpec((BLK, D), lambda i: (i, 0)),
        out_shape=jax.ShapeDtypeStruct((NP, D), jnp.float32),
    )(x, w, b)


NB = 4


def _sc_agg_body(h_hbm, src_hbm, dst_hbm,
                 sum_a, sum_b, deg_a, deg_b,
                 acc_sp, deg_sp,
                 src0, src1, src2, src3, dst0, dst1, dst2, dst3,
                 rows0, rows1, rows2, rows3,
                 dst2_v, ones_v, zflat_v, sem0, sem1, sem2, sem3):
    cid = lax.axis_index("c")
    sid = lax.axis_index("s")
    wid = cid * NS + sid

    zeros16 = jnp.zeros((16,), jnp.float32)

    @pl.loop(0, (CHUNK * D) // 16)
    def _zr(i):
        rows0[i // 8, pl.ds((i % 8) * 16, 16)] = zeros16

    @pl.loop(0, RPT // 16)
    def _zf(i):
        zflat_v[pl.ds(i * 16, 16)] = zeros16

    @pl.loop(0, DCHUNK // 16)
    def _on(i):
        ones_v[pl.ds(i * 16, 16)] = jnp.ones((16,), jnp.float32)

    for j in range(RPT // CHUNK):
        pltpu.sync_copy(rows0, acc_sp.at[pl.ds(sid * RPT + j * CHUNK, CHUNK)])
    pltpu.sync_copy(zflat_v, deg_sp.at[pl.ds(sid * RPT, RPT)])
    plsc.subcore_barrier()

    srcs = (src0, src1, src2, src3)
    dsts = (dst0, dst1, dst2, dst3)
    rows = (rows0, rows1, rows2, rows3)
    sems = (sem0, sem1, sem2, sem3)
    ebase = wid * EPW

    def _fetch(c, b):
        pltpu.sync_copy(src_hbm.at[pl.ds(ebase + c * CHUNK, CHUNK)], srcs[b])
        pltpu.sync_copy(dst_hbm.at[pl.ds(ebase + c * CHUNK, CHUNK)], dsts[b])
        pltpu.async_copy(h_hbm.at[srcs[b]], rows[b], sems[b])

    for b in range(NB):
        _fetch(b, b)

    NMAIN = (NCHUNK // NB) * NB - NB

    @pl.loop(0, NMAIN, step=NB)
    def _step(k):
        for b in range(NB):
            pltpu.make_async_copy(h_hbm.at[srcs[b]], rows[b], sems[b]).wait()
            pltpu.sync_copy(rows[b], acc_sp.at[dsts[b]], add=True)
            _fetch(k + b + NB, b)

    for c in range(NMAIN, NCHUNK):
        b = c % NB
        pltpu.make_async_copy(h_hbm.at[srcs[b]], rows[b], sems[b]).wait()
        pltpu.sync_copy(rows[b], acc_sp.at[dsts[b]], add=True)
        if c + NB < NCHUNK:
            _fetch(c + NB, b)

    @pl.loop(0, NDCHUNK)
    def _dstep(k):
        base = wid * EPW + k * DCHUNK
        pltpu.sync_copy(dst_hbm.at[pl.ds(base, DCHUNK)], dst2_v)
        pltpu.sync_copy(ones_v, deg_sp.at[dst2_v], add=True)

    plsc.subcore_barrier()

    row0 = sid * RPT

    @pl.when(cid == 0)
    def _():
        pltpu.sync_copy(acc_sp.at[pl.ds(row0, RPT)], sum_a.at[pl.ds(row0, RPT)])
        pltpu.sync_copy(deg_sp.at[pl.ds(row0, RPT)], deg_a.at[pl.ds(row0, RPT)])

    @pl.when(cid == 1)
    def _():
        pltpu.sync_copy(acc_sp.at[pl.ds(row0, RPT)], sum_b.at[pl.ds(row0, RPT)])
        pltpu.sync_copy(deg_sp.at[pl.ds(row0, RPT)], deg_b.at[pl.ds(row0, RPT)])


@functools.cache
def _make_sc_agg():
  return pl.kernel(
    _sc_agg_body,
    out_type=(
        jax.ShapeDtypeStruct((NP, D), jnp.float32),
        jax.ShapeDtypeStruct((NP, D), jnp.float32),
        jax.ShapeDtypeStruct((NP,), jnp.float32),
        jax.ShapeDtypeStruct((NP,), jnp.float32),
    ),
    mesh=plsc.VectorSubcoreMesh(
        core_axis_name="c", subcore_axis_name="s",
        num_cores=NC, num_subcores=NS),
    scratch_types=[
        pltpu.VMEM_SHARED((NP, D), jnp.float32),
        pltpu.VMEM_SHARED((NP,), jnp.float32),
        pltpu.VMEM((CHUNK,), jnp.int32),
        pltpu.VMEM((CHUNK,), jnp.int32),
        pltpu.VMEM((CHUNK,), jnp.int32),
        pltpu.VMEM((CHUNK,), jnp.int32),
        pltpu.VMEM((CHUNK,), jnp.int32),
        pltpu.VMEM((CHUNK,), jnp.int32),
        pltpu.VMEM((CHUNK,), jnp.int32),
        pltpu.VMEM((CHUNK,), jnp.int32),
        pltpu.VMEM((CHUNK, D), jnp.float32),
        pltpu.VMEM((CHUNK, D), jnp.float32),
        pltpu.VMEM((CHUNK, D), jnp.float32),
        pltpu.VMEM((CHUNK, D), jnp.float32),
        pltpu.VMEM((DCHUNK,), jnp.int32),
        pltpu.VMEM((DCHUNK,), jnp.float32),
        pltpu.VMEM((RPT,), jnp.float32),
        pltpu.SemaphoreType.DMA,
        pltpu.SemaphoreType.DMA,
        pltpu.SemaphoreType.DMA,
        pltpu.SemaphoreType.DMA,
    ],
  )


def _tail_body(h_ref, pa_ref, pb_ref, da_ref, db_ref,
               ws_ref, wn_ref, wl_ref, bs_ref, bl_ref,
               l2w_ref, l2b_ref, o_ref):
    f32 = jnp.float32
    h = h_ref[...]
    summed = pa_ref[...] + pb_ref[...]
    deg = da_ref[0] + db_ref[0]
    deg_t = lax.dot_general(deg, jnp.ones((1, D), f32),
                            (((0,), (0,)), ((), ())),
                            preferred_element_type=f32)
    h_neigh = summed / jnp.maximum(deg_t, 1.0)
    rst = (lax.dot_general(h, ws_ref[...], (((1,), (1,)), ((), ())),
                           preferred_element_type=f32)
           + lax.dot_general(h_neigh, wn_ref[...], (((1,), (1,)), ((), ())),
                             preferred_element_type=f32)
           + bs_ref[...])
    h2 = rst + h
    mu = jnp.mean(h2, axis=1, keepdims=True)
    d = h2 - mu
    var = jnp.mean(d * d, axis=1, keepdims=True)
    hn = d / jnp.sqrt(var + 1e-5) * l2w_ref[...] + l2b_ref[...]
    lin = lax.dot_general(hn, wl_ref[...], (((1,), (1,)), ((), ())),
                          preferred_element_type=f32) + bl_ref[...]
    o_ref[...] = jnp.where(lin > 0, lin, jnp.exp(jnp.minimum(lin, 0.0)) - 1.0) + hn


def _tail(h, pa, pb, da, db, w_self, w_neigh, w_lin, b_sage, b_lin,
          ln2_w, ln2_b):
    row = pl.BlockSpec((BLK, D), lambda i: (i, 0))
    mat = pl.BlockSpec((D, D), lambda i: (0, 0))
    vec = pl.BlockSpec((1, D), lambda i: (0, 0))
    dspec = pl.BlockSpec((1, 1, BLK), lambda i: (i, 0, 0))
    return pl.pallas_call(
        _tail_body,
        grid=(NBLK,),
        in_specs=[row, row, row, dspec, dspec, mat, mat, mat, vec, vec,
                  vec, vec],
        out_specs=row,
        out_shape=jax.ShapeDtypeStruct((N, D), jnp.float32),
    )(h, pa, pb, da, db, w_self, w_neigh, w_lin, b_sage, b_lin, ln2_w, ln2_b)


@jax.jit
def kernel(x, ln1_w, ln1_b, w_self, w_neigh, b_sage, ln2_w, ln2_b, w_lin,
           b_lin, edge_index):
    h = _ln1(x, ln1_w.reshape(1, D), ln1_b.reshape(1, D))
    src = edge_index[0]
    dst = edge_index[1]
    sum_a, sum_b, deg_a, deg_b = _make_sc_agg()(h, src, dst)
    return _tail(h, sum_a, sum_b,
                 deg_a.reshape(NBLK, 1, BLK), deg_b.reshape(NBLK, 1, BLK),
                 w_self, w_neigh, w_lin,
                 b_sage.reshape(1, D), b_lin.reshape(1, D),
                 ln2_w.reshape(1, D), ln2_b.reshape(1, D))


# speedup vs baseline: 1.0298x; 1.0184x over previous
import functools

import jax
import jax.numpy as jnp
from jax import lax
from jax.experimental import pallas as pl
from jax.experimental.pallas import tpu as pltpu
from jax.experimental.pallas import tpu_sc as plsc

N = 10000
E = 320000
D = 128
NP = 10240
BLK = 2048
NBLK = NP // BLK

NC = 2
NS = 16
NW = NC * NS
EPW = E // NW
CHUNK = 80
NCHUNK = EPW // CHUNK
DCHUNK = 2000
NDCHUNK = EPW // DCHUNK
RPT = NP // NS
ZR = 64


def _ln1_body(x_ref, w_ref, b_ref, o_ref):
    x = x_ref[...]
    mu = jnp.mean(x, axis=1, keepdims=True)
    d = x - mu
    var = jnp.mean(d * d, axis=1, keepdims=True)
    o_ref[...] = d / jnp.sqrt(var + 1e-5) * w_ref[...] + b_ref[...]


def _ln1(x, w, b):
    return pl.pallas_call(
        _ln1_body,
        grid=(NBLK,),
        in_specs=[
            pl.BlockSpec((BLK, D), lambda i: (i, 0)),
            pl.BlockSpec((1, D), lambda i: (0, 0)),
            pl.BlockSpec((1, D), lambda i: (0, 0)),
        ],
        out_specs=pl.BlockSpec((BLK, D), lambda i: (i, 0)),
        out_shape=jax.ShapeDtypeStruct((NP, D), jnp.float32),
    )(x, w, b)


NB = 4


def _sc_agg_body(h_hbm, src_hbm, dst_hbm,
                 sum_a, sum_b, deg_a, deg_b,
                 acc_sp, deg_sp,
                 src0, src1, src2, src3, dst0, dst1, dst2, dst3,
                 rows0, rows1, rows2, rows3,
                 dst2_v, ones_v, zflat_v, sem0, sem1, sem2, sem3):
    cid = lax.axis_index("c")
    sid = lax.axis_index("s")
    wid = cid * NS + sid

    zeros16 = jnp.zeros((16,), jnp.float32)

    @pl.loop(0, (CHUNK * D) // 16)
    def _zr(i):
        rows0[i // 8, pl.ds((i % 8) * 16, 16)] = zeros16

    @pl.loop(0, RPT // 16)
    def _zf(i):
        zflat_v[pl.ds(i * 16, 16)] = zeros16

    @pl.loop(0, DCHUNK // 16)
    def _on(i):
        ones_v[pl.ds(i * 16, 16)] = jnp.ones((16,), jnp.float32)

    for j in range(RPT // CHUNK):
        pltpu.sync_copy(rows0, acc_sp.at[pl.ds(sid * RPT + j * CHUNK, CHUNK)])
    pltpu.sync_copy(zflat_v, deg_sp.at[pl.ds(sid * RPT, RPT)])
    plsc.subcore_barrier()

    srcs = (src0, src1, src2, src3)
    dsts = (dst0, dst1, dst2, dst3)
    rows = (rows0, rows1, rows2, rows3)
    sems = (sem0, sem1, sem2, sem3)
    ebase = wid * EPW

    def _fetch(c, b):
        pltpu.sync_copy(src_hbm.at[pl.ds(ebase + c * CHUNK, CHUNK)], srcs[b])
        pltpu.sync_copy(dst_hbm.at[pl.ds(ebase + c * CHUNK, CHUNK)], dsts[b])
        pltpu.async_copy(h_hbm.at[srcs[b]], rows[b], sems[b])

    for b in range(NB):
        _fetch(b, b)

    NMAIN = (NCHUNK // NB) * NB - NB

    @pl.loop(0, NMAIN, step=NB)
    def _step(k):
        for b in range(NB):
            pltpu.make_async_copy(h_hbm.at[srcs[b]], rows[b], sems[b]).wait()
            pltpu.sync_copy(rows[b], acc_sp.at[dsts[b]], add=True)
            _fetch(k + b + NB, b)

    for c in range(NMAIN, NCHUNK):
        b = c % NB
        pltpu.make_async_copy(h_hbm.at[srcs[b]], rows[b], sems[b]).wait()
        pltpu.sync_copy(rows[b], acc_sp.at[dsts[b]], add=True)
        if c + NB < NCHUNK:
            _fetch(c + NB, b)

    @pl.loop(0, NDCHUNK)
    def _dstep(k):
        base = wid * EPW + k * DCHUNK
        pltpu.sync_copy(dst_hbm.at[pl.ds(base, DCHUNK)], dst2_v)
        pltpu.sync_copy(ones_v, deg_sp.at[dst2_v], add=True)

    plsc.subcore_barrier()

    row0 = sid * RPT

    @pl.when(cid == 0)
    def _():
        pltpu.sync_copy(acc_sp.at[pl.ds(row0, RPT)], sum_a.at[pl.ds(row0, RPT)])
        pltpu.sync_copy(deg_sp.at[pl.ds(row0, RPT)], deg_a.at[pl.ds(row0, RPT)])

    @pl.when(cid == 1)
    def _():
        pltpu.sync_copy(acc_sp.at[pl.ds(row0, RPT)], sum_b.at[pl.ds(row0, RPT)])
        pltpu.sync_copy(deg_sp.at[pl.ds(row0, RPT)], deg_b.at[pl.ds(row0, RPT)])


@functools.cache
def _make_sc_agg():
  return pl.kernel(
    _sc_agg_body,
    out_type=(
        jax.ShapeDtypeStruct((NP, D), jnp.float32),
        jax.ShapeDtypeStruct((NP, D), jnp.float32),
        jax.ShapeDtypeStruct((NP,), jnp.float32),
        jax.ShapeDtypeStruct((NP,), jnp.float32),
    ),
    mesh=plsc.VectorSubcoreMesh(
        core_axis_name="c", subcore_axis_name="s",
        num_cores=NC, num_subcores=NS),
    scratch_types=[
        pltpu.VMEM_SHARED((NP, D), jnp.float32),
        pltpu.VMEM_SHARED((NP,), jnp.float32),
        pltpu.VMEM((CHUNK,), jnp.int32),
        pltpu.VMEM((CHUNK,), jnp.int32),
        pltpu.VMEM((CHUNK,), jnp.int32),
        pltpu.VMEM((CHUNK,), jnp.int32),
        pltpu.VMEM((CHUNK,), jnp.int32),
        pltpu.VMEM((CHUNK,), jnp.int32),
        pltpu.VMEM((CHUNK,), jnp.int32),
        pltpu.VMEM((CHUNK,), jnp.int32),
        pltpu.VMEM((CHUNK, D), jnp.float32),
        pltpu.VMEM((CHUNK, D), jnp.float32),
        pltpu.VMEM((CHUNK, D), jnp.float32),
        pltpu.VMEM((CHUNK, D), jnp.float32),
        pltpu.VMEM((DCHUNK,), jnp.int32),
        pltpu.VMEM((DCHUNK,), jnp.float32),
        pltpu.VMEM((RPT,), jnp.float32),
        pltpu.SemaphoreType.DMA,
        pltpu.SemaphoreType.DMA,
        pltpu.SemaphoreType.DMA,
        pltpu.SemaphoreType.DMA,
    ],
  )


def _tail_body(h_ref, pa_ref, pb_ref, da_ref, db_ref,
               ws_ref, wn_ref, wl_ref, bs_ref, bl_ref,
               l2w_ref, l2b_ref, o_ref):
    f32 = jnp.float32
    h = h_ref[...]
    summed = pa_ref[...] + pb_ref[...]
    deg = da_ref[0] + db_ref[0]
    deg_t = lax.dot_general(deg, jnp.ones((1, D), f32),
                            (((0,), (0,)), ((), ())),
                            preferred_element_type=f32)
    h_neigh = summed / jnp.maximum(deg_t, 1.0)
    rst = (lax.dot_general(h, ws_ref[...], (((1,), (1,)), ((), ())),
                           preferred_element_type=f32)
           + lax.dot_general(h_neigh, wn_ref[...], (((1,), (1,)), ((), ())),
                             preferred_element_type=f32)
           + bs_ref[...])
    h2 = rst + h
    mu = jnp.mean(h2, axis=1, keepdims=True)
    d = h2 - mu
    var = jnp.mean(d * d, axis=1, keepdims=True)
    hn = d / jnp.sqrt(var + 1e-5) * l2w_ref[...] + l2b_ref[...]
    lin = lax.dot_general(hn, wl_ref[...], (((1,), (1,)), ((), ())),
                          preferred_element_type=f32) + bl_ref[...]
    o_ref[...] = jnp.where(lin > 0, lin, jnp.exp(jnp.minimum(lin, 0.0)) - 1.0) + hn


def _tail(h, pa, pb, da, db, w_self, w_neigh, w_lin, b_sage, b_lin,
          ln2_w, ln2_b):
    row = pl.BlockSpec((BLK, D), lambda i: (i, 0))
    mat = pl.BlockSpec((D, D), lambda i: (0, 0))
    vec = pl.BlockSpec((1, D), lambda i: (0, 0))
    dspec = pl.BlockSpec((1, 1, BLK), lambda i: (i, 0, 0))
    return pl.pallas_call(
        _tail_body,
        grid=(NBLK,),
        in_specs=[row, row, row, dspec, dspec, mat, mat, mat, vec, vec,
                  vec, vec],
        out_specs=row,
        out_shape=jax.ShapeDtypeStruct((N, D), jnp.float32),
    )(h, pa, pb, da, db, w_self, w_neigh, w_lin, b_sage, b_lin, ln2_w, ln2_b)


@jax.jit
def kernel(x, ln1_w, ln1_b, w_self, w_neigh, b_sage, ln2_w, ln2_b, w_lin,
           b_lin, edge_index):
    h = _ln1(x, ln1_w.reshape(1, D), ln1_b.reshape(1, D))
    src = edge_index[0]
    dst = edge_index[1]
    sum_a, sum_b, deg_a, deg_b = _make_sc_agg()(h, src, dst)
    return _tail(h, sum_a, sum_b,
                 deg_a.reshape(NBLK, 1, BLK), deg_b.reshape(NBLK, 1, BLK),
                 w_self, w_neigh, w_lin,
                 b_sage.reshape(1, D), b_lin.reshape(1, D),
                 ln2_w.reshape(1, D), ln2_b.reshape(1, D))


# speedup vs baseline: 1.0355x; 1.0055x over previous
import functools

import jax
import jax.numpy as jnp
from jax import lax
from jax.experimental import pallas as pl
from jax.experimental.pallas import tpu as pltpu
from jax.experimental.pallas import tpu_sc as plsc

N = 10000
E = 320000
D = 128
NP = 10240
BLK = 5120
NBLK = NP // BLK

NC = 2
NS = 16
NW = NC * NS
EPW = E // NW
CHUNK = 80
NCHUNK = EPW // CHUNK
DCHUNK = 2000
NDCHUNK = EPW // DCHUNK
RPT = NP // NS
ZR = 64


def _ln1_body(x_ref, w_ref, b_ref, o_ref):
    x = x_ref[...]
    mu = jnp.mean(x, axis=1, keepdims=True)
    d = x - mu
    var = jnp.mean(d * d, axis=1, keepdims=True)
    o_ref[...] = d / jnp.sqrt(var + 1e-5) * w_ref[...] + b_ref[...]


def _ln1(x, w, b):
    return pl.pallas_call(
        _ln1_body,
        grid=(NBLK,),
        in_specs=[
            pl.BlockSpec((BLK, D), lambda i: (i, 0)),
            pl.BlockSpec((1, D), lambda i: (0, 0)),
            pl.BlockSpec((1, D), lambda i: (0, 0)),
        ],
        out_specs=pl.BlockSpec((BLK, D), lambda i: (i, 0)),
        out_shape=jax.ShapeDtypeStruct((NP, D), jnp.float32),
    )(x, w, b)


NB = 4


def _sc_agg_body(h_hbm, src_hbm, dst_hbm,
                 sum_a, sum_b, deg_a, deg_b,
                 acc_sp, deg_sp,
                 src0, src1, src2, src3, dst0, dst1, dst2, dst3,
                 rows0, rows1, rows2, rows3,
                 dst2_v, ones_v, zflat_v, sem0, sem1, sem2, sem3):
    cid = lax.axis_index("c")
    sid = lax.axis_index("s")
    wid = cid * NS + sid

    zeros16 = jnp.zeros((16,), jnp.float32)

    @pl.loop(0, (CHUNK * D) // 16)
    def _zr(i):
        rows0[i // 8, pl.ds((i % 8) * 16, 16)] = zeros16

    @pl.loop(0, RPT // 16)
    def _zf(i):
        zflat_v[pl.ds(i * 16, 16)] = zeros16

    @pl.loop(0, DCHUNK // 16)
    def _on(i):
        ones_v[pl.ds(i * 16, 16)] = jnp.ones((16,), jnp.float32)

    for j in range(RPT // CHUNK):
        pltpu.sync_copy(rows0, acc_sp.at[pl.ds(sid * RPT + j * CHUNK, CHUNK)])
    pltpu.sync_copy(zflat_v, deg_sp.at[pl.ds(sid * RPT, RPT)])
    plsc.subcore_barrier()

    srcs = (src0, src1, src2, src3)
    dsts = (dst0, dst1, dst2, dst3)
    rows = (rows0, rows1, rows2, rows3)
    sems = (sem0, sem1, sem2, sem3)
    ebase = wid * EPW

    def _fetch(c, b):
        pltpu.sync_copy(src_hbm.at[pl.ds(ebase + c * CHUNK, CHUNK)], srcs[b])
        pltpu.sync_copy(dst_hbm.at[pl.ds(ebase + c * CHUNK, CHUNK)], dsts[b])
        pltpu.async_copy(h_hbm.at[srcs[b]], rows[b], sems[b])

    for b in range(NB):
        _fetch(b, b)

    NMAIN = (NCHUNK // NB) * NB - NB

    @pl.loop(0, NMAIN, step=NB)
    def _step(k):
        for b in range(NB):
            pltpu.make_async_copy(h_hbm.at[srcs[b]], rows[b], sems[b]).wait()
            pltpu.sync_copy(rows[b], acc_sp.at[dsts[b]], add=True)
            _fetch(k + b + NB, b)

    for c in range(NMAIN, NCHUNK):
        b = c % NB
        pltpu.make_async_copy(h_hbm.at[srcs[b]], rows[b], sems[b]).wait()
        pltpu.sync_copy(rows[b], acc_sp.at[dsts[b]], add=True)
        if c + NB < NCHUNK:
            _fetch(c + NB, b)

    @pl.loop(0, NDCHUNK)
    def _dstep(k):
        base = wid * EPW + k * DCHUNK
        pltpu.sync_copy(dst_hbm.at[pl.ds(base, DCHUNK)], dst2_v)
        pltpu.sync_copy(ones_v, deg_sp.at[dst2_v], add=True)

    plsc.subcore_barrier()

    row0 = sid * RPT

    @pl.when(cid == 0)
    def _():
        pltpu.sync_copy(acc_sp.at[pl.ds(row0, RPT)], sum_a.at[pl.ds(row0, RPT)])
        pltpu.sync_copy(deg_sp.at[pl.ds(row0, RPT)], deg_a.at[pl.ds(row0, RPT)])

    @pl.when(cid == 1)
    def _():
        pltpu.sync_copy(acc_sp.at[pl.ds(row0, RPT)], sum_b.at[pl.ds(row0, RPT)])
        pltpu.sync_copy(deg_sp.at[pl.ds(row0, RPT)], deg_b.at[pl.ds(row0, RPT)])


@functools.cache
def _make_sc_agg():
  return pl.kernel(
    _sc_agg_body,
    out_type=(
        jax.ShapeDtypeStruct((NP, D), jnp.float32),
        jax.ShapeDtypeStruct((NP, D), jnp.float32),
        jax.ShapeDtypeStruct((NP,), jnp.float32),
        jax.ShapeDtypeStruct((NP,), jnp.float32),
    ),
    mesh=plsc.VectorSubcoreMesh(
        core_axis_name="c", subcore_axis_name="s",
        num_cores=NC, num_subcores=NS),
    scratch_types=[
        pltpu.VMEM_SHARED((NP, D), jnp.float32),
        pltpu.VMEM_SHARED((NP,), jnp.float32),
        pltpu.VMEM((CHUNK,), jnp.int32),
        pltpu.VMEM((CHUNK,), jnp.int32),
        pltpu.VMEM((CHUNK,), jnp.int32),
        pltpu.VMEM((CHUNK,), jnp.int32),
        pltpu.VMEM((CHUNK,), jnp.int32),
        pltpu.VMEM((CHUNK,), jnp.int32),
        pltpu.VMEM((CHUNK,), jnp.int32),
        pltpu.VMEM((CHUNK,), jnp.int32),
        pltpu.VMEM((CHUNK, D), jnp.float32),
        pltpu.VMEM((CHUNK, D), jnp.float32),
        pltpu.VMEM((CHUNK, D), jnp.float32),
        pltpu.VMEM((CHUNK, D), jnp.float32),
        pltpu.VMEM((DCHUNK,), jnp.int32),
        pltpu.VMEM((DCHUNK,), jnp.float32),
        pltpu.VMEM((RPT,), jnp.float32),
        pltpu.SemaphoreType.DMA,
        pltpu.SemaphoreType.DMA,
        pltpu.SemaphoreType.DMA,
        pltpu.SemaphoreType.DMA,
    ],
  )


def _tail_body(h_ref, pa_ref, pb_ref, da_ref, db_ref,
               ws_ref, wn_ref, wl_ref, bs_ref, bl_ref,
               l2w_ref, l2b_ref, o_ref):
    f32 = jnp.float32
    h = h_ref[...]
    summed = pa_ref[...] + pb_ref[...]
    deg = da_ref[0] + db_ref[0]
    deg_t = lax.dot_general(deg, jnp.ones((1, D), f32),
                            (((0,), (0,)), ((), ())),
                            preferred_element_type=f32)
    h_neigh = summed / jnp.maximum(deg_t, 1.0)
    rst = (lax.dot_general(h, ws_ref[...], (((1,), (1,)), ((), ())),
                           preferred_element_type=f32)
           + lax.dot_general(h_neigh, wn_ref[...], (((1,), (1,)), ((), ())),
                             preferred_element_type=f32)
           + bs_ref[...])
    h2 = rst + h
    mu = jnp.mean(h2, axis=1, keepdims=True)
    d = h2 - mu
    var = jnp.mean(d * d, axis=1, keepdims=True)
    hn = d / jnp.sqrt(var + 1e-5) * l2w_ref[...] + l2b_ref[...]
    lin = lax.dot_general(hn, wl_ref[...], (((1,), (1,)), ((), ())),
                          preferred_element_type=f32) + bl_ref[...]
    o_ref[...] = jnp.where(lin > 0, lin, jnp.exp(jnp.minimum(lin, 0.0)) - 1.0) + hn


def _tail(h, pa, pb, da, db, w_self, w_neigh, w_lin, b_sage, b_lin,
          ln2_w, ln2_b):
    row = pl.BlockSpec((BLK, D), lambda i: (i, 0))
    mat = pl.BlockSpec((D, D), lambda i: (0, 0))
    vec = pl.BlockSpec((1, D), lambda i: (0, 0))
    dspec = pl.BlockSpec((1, 1, BLK), lambda i: (i, 0, 0))
    return pl.pallas_call(
        _tail_body,
        grid=(NBLK,),
        in_specs=[row, row, row, dspec, dspec, mat, mat, mat, vec, vec,
                  vec, vec],
        out_specs=row,
        out_shape=jax.ShapeDtypeStruct((N, D), jnp.float32),
    )(h, pa, pb, da, db, w_self, w_neigh, w_lin, b_sage, b_lin, ln2_w, ln2_b)


@jax.jit
def kernel(x, ln1_w, ln1_b, w_self, w_neigh, b_sage, ln2_w, ln2_b, w_lin,
           b_lin, edge_index):
    h = _ln1(x, ln1_w.reshape(1, D), ln1_b.reshape(1, D))
    src = edge_index[0]
    dst = edge_index[1]
    sum_a, sum_b, deg_a, deg_b = _make_sc_agg()(h, src, dst)
    return _tail(h, sum_a, sum_b,
                 deg_a.reshape(NBLK, 1, BLK), deg_b.reshape(NBLK, 1, BLK),
                 w_self, w_neigh, w_lin,
                 b_sage.reshape(1, D), b_lin.reshape(1, D),
                 ln2_w.reshape(1, D), ln2_b.reshape(1, D))


# speedup vs baseline: 1.3036x; 1.2589x over previous
import functools

import jax
import jax.numpy as jnp
from jax import lax
from jax.experimental import pallas as pl
from jax.experimental.pallas import tpu as pltpu
from jax.experimental.pallas import tpu_sc as plsc

N = 10000
E = 320000
D = 128
NP = 10240
BLK = 5120
NBLK = NP // BLK

NC = 2
NS = 16
NW = NC * NS
EPW = E // NW
CHUNK = 80
NCHUNK = EPW // CHUNK
DCHUNK = 400
NDCHUNK = EPW // DCHUNK
RPT = NP // NS
ZR = 64


def _ln1_body(x_ref, w_ref, b_ref, o_ref):
    x = x_ref[...]
    mu = jnp.mean(x, axis=1, keepdims=True)
    d = x - mu
    var = jnp.mean(d * d, axis=1, keepdims=True)
    o_ref[...] = d / jnp.sqrt(var + 1e-5) * w_ref[...] + b_ref[...]


def _ln1(x, w, b):
    return pl.pallas_call(
        _ln1_body,
        grid=(NBLK,),
        in_specs=[
            pl.BlockSpec((BLK, D), lambda i: (i, 0)),
            pl.BlockSpec((1, D), lambda i: (0, 0)),
            pl.BlockSpec((1, D), lambda i: (0, 0)),
        ],
        out_specs=pl.BlockSpec((BLK, D), lambda i: (i, 0)),
        out_shape=jax.ShapeDtypeStruct((NP, D), jnp.float32),
    )(x, w, b)


NB = 2


def _sc_agg_body(h_hbm, src_hbm, dst_hbm, dstr_hbm,
                 sum_a, sum_b, deg_a, deg_b,
                 acc_sp, deg_sp,
                 src_flat, dst_big, rows0, rows1,
                 dst2_v, ones_v, zflat_v, sem0, sem1):
    cid = lax.axis_index("c")
    sid = lax.axis_index("s")
    wid = cid * NS + sid

    zeros16 = jnp.zeros((16,), jnp.float32)

    @pl.loop(0, (CHUNK * D) // 16)
    def _zr(i):
        rows0[i // 8, pl.ds((i % 8) * 16, 16)] = zeros16

    @pl.loop(0, (RPT // 2) // 16)
    def _zf(i):
        zflat_v[pl.ds(i * 16, 16)] = zeros16

    @pl.loop(0, DCHUNK // 16)
    def _on(i):
        ones_v[pl.ds(i * 16, 16)] = jnp.ones((16,), jnp.float32)

    for j in range(RPT // CHUNK):
        pltpu.sync_copy(rows0, acc_sp.at[pl.ds(sid * RPT + j * CHUNK, CHUNK)])
    for j in range(2):
        pltpu.sync_copy(zflat_v,
                        deg_sp.at[pl.ds(sid * RPT + j * (RPT // 2), RPT // 2)])
    plsc.subcore_barrier()

    rows = (rows0, rows1)
    sems = (sem0, sem1)
    ebase = wid * EPW

    pltpu.sync_copy(src_hbm.at[pl.ds(ebase, EPW)], src_flat)
    pltpu.sync_copy(dstr_hbm.at[wid], dst_big)

    def _gather(c, b):
        pltpu.async_copy(
            h_hbm.at[src_flat.at[pl.ds(c * CHUNK, CHUNK)]], rows[b], sems[b])

    def _wait(b):
        pltpu.make_async_copy(
            h_hbm.at[src_flat.at[pl.ds(0, CHUNK)]], rows[b], sems[b]).wait()

    for b in range(NB):
        _gather(b, b)

    NMAIN = (NCHUNK // NB) * NB - NB

    @pl.loop(0, NMAIN, step=NB)
    def _step(k):
        for b in range(NB):
            _wait(b)
            pltpu.sync_copy(rows[b], acc_sp.at[dst_big.at[k + b]], add=True)
            _gather(k + b + NB, b)

    for c in range(NMAIN, NCHUNK):
        b = c % NB
        _wait(b)
        pltpu.sync_copy(rows[b], acc_sp.at[dst_big.at[c]], add=True)
        if c + NB < NCHUNK:
            _gather(c + NB, b)

    @pl.loop(0, NDCHUNK)
    def _dstep(k):
        base = wid * EPW + k * DCHUNK
        pltpu.sync_copy(dst_hbm.at[pl.ds(base, DCHUNK)], dst2_v)
        pltpu.sync_copy(ones_v, deg_sp.at[dst2_v], add=True)

    plsc.subcore_barrier()

    row0 = sid * RPT

    @pl.when(cid == 0)
    def _():
        pltpu.sync_copy(acc_sp.at[pl.ds(row0, RPT)], sum_a.at[pl.ds(row0, RPT)])
        pltpu.sync_copy(deg_sp.at[pl.ds(row0, RPT)], deg_a.at[pl.ds(row0, RPT)])

    @pl.when(cid == 1)
    def _():
        pltpu.sync_copy(acc_sp.at[pl.ds(row0, RPT)], sum_b.at[pl.ds(row0, RPT)])
        pltpu.sync_copy(deg_sp.at[pl.ds(row0, RPT)], deg_b.at[pl.ds(row0, RPT)])


@functools.cache
def _make_sc_agg():
  return pl.kernel(
    _sc_agg_body,
    out_type=(
        jax.ShapeDtypeStruct((NP, D), jnp.float32),
        jax.ShapeDtypeStruct((NP, D), jnp.float32),
        jax.ShapeDtypeStruct((NP,), jnp.float32),
        jax.ShapeDtypeStruct((NP,), jnp.float32),
    ),
    mesh=plsc.VectorSubcoreMesh(
        core_axis_name="c", subcore_axis_name="s",
        num_cores=NC, num_subcores=NS),
    scratch_types=[
        pltpu.VMEM_SHARED((NP, D), jnp.float32),
        pltpu.VMEM_SHARED((NP,), jnp.float32),
        pltpu.VMEM((EPW,), jnp.int32),
        pltpu.VMEM((NCHUNK, CHUNK), jnp.int32),
        pltpu.VMEM((CHUNK, D), jnp.float32),
        pltpu.VMEM((CHUNK, D), jnp.float32),
        pltpu.VMEM((DCHUNK,), jnp.int32),
        pltpu.VMEM((DCHUNK,), jnp.float32),
        pltpu.VMEM((RPT // 2,), jnp.float32),
        pltpu.SemaphoreType.DMA,
        pltpu.SemaphoreType.DMA,
    ],
  )


def _tail_body(h_ref, pa_ref, pb_ref, da_ref, db_ref,
               ws_ref, wn_ref, wl_ref, bs_ref, bl_ref,
               l2w_ref, l2b_ref, o_ref):
    f32 = jnp.float32
    h = h_ref[...]
    summed = pa_ref[...] + pb_ref[...]
    deg = da_ref[0] + db_ref[0]
    deg_t = lax.dot_general(deg, jnp.ones((1, D), f32),
                            (((0,), (0,)), ((), ())),
                            preferred_element_type=f32)
    h_neigh = summed / jnp.maximum(deg_t, 1.0)
    rst = (lax.dot_general(h, ws_ref[...], (((1,), (1,)), ((), ())),
                           preferred_element_type=f32)
           + lax.dot_general(h_neigh, wn_ref[...], (((1,), (1,)), ((), ())),
                             preferred_element_type=f32)
           + bs_ref[...])
    h2 = rst + h
    mu = jnp.mean(h2, axis=1, keepdims=True)
    d = h2 - mu
    var = jnp.mean(d * d, axis=1, keepdims=True)
    hn = d / jnp.sqrt(var + 1e-5) * l2w_ref[...] + l2b_ref[...]
    lin = lax.dot_general(hn, wl_ref[...], (((1,), (1,)), ((), ())),
                          preferred_element_type=f32) + bl_ref[...]
    o_ref[...] = jnp.where(lin > 0, lin, jnp.exp(jnp.minimum(lin, 0.0)) - 1.0) + hn


def _tail(h, pa, pb, da, db, w_self, w_neigh, w_lin, b_sage, b_lin,
          ln2_w, ln2_b):
    row = pl.BlockSpec((BLK, D), lambda i: (i, 0))
    mat = pl.BlockSpec((D, D), lambda i: (0, 0))
    vec = pl.BlockSpec((1, D), lambda i: (0, 0))
    dspec = pl.BlockSpec((1, 1, BLK), lambda i: (i, 0, 0))
    return pl.pallas_call(
        _tail_body,
        grid=(NBLK,),
        in_specs=[row, row, row, dspec, dspec, mat, mat, mat, vec, vec,
                  vec, vec],
        out_specs=row,
        out_shape=jax.ShapeDtypeStruct((N, D), jnp.float32),
    )(h, pa, pb, da, db, w_self, w_neigh, w_lin, b_sage, b_lin, ln2_w, ln2_b)


@jax.jit
def kernel(x, ln1_w, ln1_b, w_self, w_neigh, b_sage, ln2_w, ln2_b, w_lin,
           b_lin, edge_index):
    h = _ln1(x, ln1_w.reshape(1, D), ln1_b.reshape(1, D))
    src = edge_index[0]
    dst = edge_index[1]
    sum_a, sum_b, deg_a, deg_b = _make_sc_agg()(
        h, src, dst, dst.reshape(NW, NCHUNK, CHUNK))
    return _tail(h, sum_a, sum_b,
                 deg_a.reshape(NBLK, 1, BLK), deg_b.reshape(NBLK, 1, BLK),
                 w_self, w_neigh, w_lin,
                 b_sage.reshape(1, D), b_lin.reshape(1, D),
                 ln2_w.reshape(1, D), ln2_b.reshape(1, D))
